# Initial kernel scaffold; baseline (speedup 1.0000x reference)
import jax
import jax.numpy as jnp
from jax.experimental import pallas as pl


def kernel(encoded_ast_nodes, symbol_leaf_nodes_indices, symbol_leaf_symbol_idx):
    raise NotImplementedError("write your pallas kernel here")


# speedup vs baseline: 1.5094x; 1.5094x over previous
import functools

import jax
import jax.numpy as jnp
from jax import lax
from jax.experimental import pallas as pl
from jax.experimental.pallas import tpu as pltpu
from jax.experimental.pallas import tpu_sc as plsc

_B = 100000
_D = 256
_V = 200000
_CH = 224
_NFULL = _B // _CH
_TAIL = _B - _NFULL * _CH
_NCHUNKS = _NFULL + 1

_INFO = plsc.get_sparse_core_info()
_NC = _INFO.num_cores
_NW = _INFO.num_cores * _INFO.num_subcores
_ITERS = -(-_NCHUNKS // _NW)

_MESH = plsc.VectorSubcoreMesh(core_axis_name="c", subcore_axis_name="s")


@functools.partial(
    pl.kernel,
    mesh=_MESH,
    out_type=jax.ShapeDtypeStruct((_B, _D), jnp.float32),
    scratch_types=[
        pltpu.VMEM((_CH,), jnp.int32),
        pltpu.VMEM((_CH, _D), jnp.float32),
        pltpu.SemaphoreType.DMA,
    ],
)
def _gather_sc(table_hbm, idx_hbm, out_hbm, idx_v, rows_v, sem):
    wid = lax.axis_index("s") * _NC + lax.axis_index("c")

    def body(i, carry):
        c = wid + i * _NW

        @pl.when(c < _NFULL)
        def _full():
            base = c * _CH
            pltpu.sync_copy(idx_hbm.at[pl.ds(base, _CH)], idx_v)
            pltpu.async_copy(table_hbm.at[idx_v], rows_v, sem).wait()
            pltpu.sync_copy(rows_v, out_hbm.at[pl.ds(base, _CH)])

        @pl.when(c == _NFULL)
        def _tail():
            base = _NFULL * _CH
            pltpu.sync_copy(idx_hbm.at[pl.ds(base, _TAIL)],
                            idx_v.at[pl.ds(0, _TAIL)])
            pltpu.async_copy(table_hbm.at[idx_v], rows_v, sem).wait()
            pltpu.sync_copy(rows_v.at[pl.ds(0, _TAIL)],
                            out_hbm.at[pl.ds(base, _TAIL)])

        return carry

    lax.fori_loop(0, _ITERS, body, 0)


def kernel(encoded_ast_nodes, symbol_leaf_nodes_indices, symbol_leaf_symbol_idx):
    out = _gather_sc(encoded_ast_nodes, symbol_leaf_nodes_indices)
    return (out, symbol_leaf_symbol_idx)


# speedup vs baseline: 1.5827x; 1.0486x over previous
import jax
import jax.numpy as jnp
from jax import lax
from jax.experimental import pallas as pl
from jax.experimental.pallas import tpu as pltpu
from jax.experimental.pallas import tpu_sc as plsc

_B = 100000
_D = 256
_CH = 200
_NCHUNKS = _B // _CH
_ITERS = -(-_NCHUNKS // 32)

_INFO = plsc.get_sparse_core_info()
_NC = _INFO.num_cores
_NW = _INFO.num_cores * _INFO.num_subcores

_MESH = plsc.VectorSubcoreMesh(core_axis_name="c", subcore_axis_name="s")


@pl.kernel(
    mesh=_MESH,
    out_type=jax.ShapeDtypeStruct((_B, _D), jnp.float32),
    scratch_types=[
        pltpu.VMEM((_CH,), jnp.int32),
        pltpu.VMEM((_CH,), jnp.int32),
        pltpu.VMEM((_CH, _D), jnp.float32),
        pltpu.VMEM((_CH, _D), jnp.float32),
        pltpu.SemaphoreType.DMA,
        pltpu.SemaphoreType.DMA,
        pltpu.SemaphoreType.DMA,
        pltpu.SemaphoreType.DMA,
        pltpu.SemaphoreType.DMA,
    ],
)
def _gather_sc(table_hbm, idx_hbm, out_hbm, idx_v0, idx_v1, rows_v0, rows_v1,
               sem_i0, sem_i1, sem_g, sem_s0, sem_s1):
    wid = lax.axis_index("s") * _NC + lax.axis_index("c")
    idx_v = (idx_v0, idx_v1)
    rows_v = (rows_v0, rows_v1)
    sem_i = (sem_i0, sem_i1)
    sem_s = (sem_s0, sem_s1)

    def c_of(i):
        return wid + i * _NW

    def start_idx(i):
        b = i % 2
        pltpu.async_copy(idx_hbm.at[pl.ds(c_of(i) * _CH, _CH)],
                         idx_v[b], sem_i[b])

    start_idx(0)
    for i in range(_ITERS):
        b = i % 2
        c = c_of(i)

        if i + 1 < _ITERS:
            @pl.when(c + _NW < _NCHUNKS)
            def _prefetch(i=i):
                start_idx(i + 1)

        @pl.when(c < _NCHUNKS)
        def _chunk(b=b, i=i, c=c):
            pltpu.make_async_copy(idx_hbm.at[pl.ds(c * _CH, _CH)],
                                  idx_v[b], sem_i[b]).wait()
            if i >= 2:
                pltpu.make_async_copy(
                    rows_v[b], out_hbm.at[pl.ds((c - 2 * _NW) * _CH, _CH)],
                    sem_s[b]).wait()
            pltpu.async_copy(table_hbm.at[idx_v[b]], rows_v[b],
                             sem_g).wait()
            pltpu.async_copy(rows_v[b], out_hbm.at[pl.ds(c * _CH, _CH)],
                             sem_s[b])

    for i in range(_ITERS - 2, _ITERS):
        b = i % 2
        c = c_of(i)

        @pl.when(c < _NCHUNKS)
        def _drain(b=b, c=c):
            pltpu.make_async_copy(rows_v[b],
                                  out_hbm.at[pl.ds(c * _CH, _CH)],
                                  sem_s[b]).wait()


def kernel(encoded_ast_nodes, symbol_leaf_nodes_indices, symbol_leaf_symbol_idx):
    out = _gather_sc(encoded_ast_nodes, symbol_leaf_nodes_indices)
    return (out, symbol_leaf_symbol_idx)


# speedup vs baseline: 1.6412x; 1.0370x over previous
import jax
import jax.numpy as jnp
from jax import lax
from jax.experimental import pallas as pl
from jax.experimental.pallas import tpu as pltpu
from jax.experimental.pallas import tpu_sc as plsc

_B = 100000
_D = 256
_CH = 160
_NCHUNKS = _B // _CH
_ITERS = -(-_NCHUNKS // 32)
_NRB = 3
_NIB = 4

_INFO = plsc.get_sparse_core_info()
_NC = _INFO.num_cores
_NW = _INFO.num_cores * _INFO.num_subcores

_MESH = plsc.VectorSubcoreMesh(core_axis_name="c", subcore_axis_name="s")


@pl.kernel(
    mesh=_MESH,
    out_type=jax.ShapeDtypeStruct((_B, _D), jnp.float32),
    scratch_types=(
        [pltpu.VMEM((_CH,), jnp.int32) for _ in range(_NIB)]
        + [pltpu.VMEM((_CH, _D), jnp.float32) for _ in range(_NRB)]
        + [pltpu.SemaphoreType.DMA for _ in range(_NIB + 2 * _NRB)]
    ),
)
def _gather_sc(table_hbm, idx_hbm, out_hbm, *scratch):
    idx_v = scratch[:_NIB]
    rows_v = scratch[_NIB:_NIB + _NRB]
    sem_i = scratch[_NIB + _NRB:2 * _NIB + _NRB]
    sem_g = scratch[2 * _NIB + _NRB:2 * _NIB + 2 * _NRB]
    sem_s = scratch[2 * _NIB + 2 * _NRB:]

    wid = lax.axis_index("s") * _NC + lax.axis_index("c")

    def c_of(i):
        return wid + i * _NW

    def idx_copy(i):
        b = i % _NIB
        return pltpu.make_async_copy(idx_hbm.at[pl.ds(c_of(i) * _CH, _CH)],
                                     idx_v[b], sem_i[b])

    def gather_copy(i):
        b = i % _NRB
        return pltpu.make_async_copy(table_hbm.at[idx_v[i % _NIB]],
                                     rows_v[b], sem_g[b])

    def store_copy(i):
        b = i % _NRB
        return pltpu.make_async_copy(rows_v[b],
                                     out_hbm.at[pl.ds(c_of(i) * _CH, _CH)],
                                     sem_s[b])

    idx_copy(0).start()
    idx_copy(1).start()

    for i in range(_ITERS + 1):
        if i < _ITERS:
            @pl.when(c_of(i) < _NCHUNKS)
            def _launch(i=i):
                idx_copy(i).wait()
                if i >= _NRB:
                    store_copy(i - _NRB).wait()
                gather_copy(i).start()

            if i + 2 < _ITERS:
                @pl.when(c_of(i + 2) < _NCHUNKS)
                def _prefetch(i=i):
                    idx_copy(i + 2).start()

        if i >= 1:
            @pl.when(c_of(i - 1) < _NCHUNKS)
            def _complete(i=i):
                gather_copy(i - 1).wait()
                store_copy(i - 1).start()

    for i in range(_ITERS - _NRB, _ITERS):
        @pl.when(c_of(i) < _NCHUNKS)
        def _drain(i=i):
            store_copy(i).wait()


def kernel(encoded_ast_nodes, symbol_leaf_nodes_indices, symbol_leaf_symbol_idx):
    out = _gather_sc(encoded_ast_nodes, symbol_leaf_nodes_indices)
    return (out, symbol_leaf_symbol_idx)
